# baseline (speedup 1.0000x reference)
import functools
import jax
import jax.numpy as jnp
from jax import lax
from jax.experimental import pallas as pl
from jax.experimental.pallas import tpu as pltpu
from jax.experimental.pallas import tpu_sc as plsc

N = 10000
E = 160000
D = 256
H = 128
E_P = 163840
EPT_C = E_P // 16
EPT_AD = E_P // 32
N_ACC = 10112
ROWS_PT = N_ACC // 16
CHUNK = 128
NCHUNK = EPT_C // CHUNK
NBUF = 2

_MESH = plsc.VectorSubcoreMesh(core_axis_name="c", subcore_axis_name="s")


@functools.partial(
    pl.kernel, mesh=_MESH,
    out_type=jax.ShapeDtypeStruct((32, N_ACC), jnp.float32),
    scratch_types=[
        pltpu.VMEM((EPT_AD,), jnp.int32),
        pltpu.VMEM((N_ACC,), jnp.float32),
    ],
    compiler_params=pltpu.CompilerParams(needs_layout_passes=False),
)
def _deg_kernel(dst_hbm, zeros_hbm, out_hbm, dst_v, acc_v):
    c = lax.axis_index("c")
    s = lax.axis_index("s")
    wid = s * 2 + c
    pltpu.sync_copy(zeros_hbm, acc_v)
    pltpu.sync_copy(dst_hbm.at[pl.ds(wid * EPT_AD, EPT_AD)], dst_v)
    ones = jnp.full((16,), 1.0, jnp.float32)

    def body(i, carry):
        idx = dst_v[pl.ds(i * 16, 16)]
        plsc.addupdate_scatter(acc_v, [idx], ones)
        return carry

    lax.fori_loop(0, EPT_AD // 16, body, 0)
    pltpu.sync_copy(acc_v, out_hbm.at[wid])


@functools.partial(
    pl.kernel, mesh=_MESH,
    out_type=jax.ShapeDtypeStruct((2 * N, H), jnp.float32),
    scratch_types=[
        pltpu.VMEM((NCHUNK // 2, CHUNK), jnp.int32),
        pltpu.VMEM((NCHUNK // 2, CHUNK), jnp.int32),
        [pltpu.VMEM((CHUNK, H), jnp.float32)] * NBUF,
        [pltpu.SemaphoreType.DMA] * NBUF,
        pltpu.VMEM_SHARED((N_ACC, H), jnp.float32),
    ],
)
def _agg_kernel(srcb_hbm, dstp_hbm, g_hbm, zrows_hbm, out_hbm,
                idx_s, idx_d, rows, gsem, acc_sh):
    c = lax.axis_index("c")
    s = lax.axis_index("s")
    pltpu.sync_copy(zrows_hbm, acc_sh.at[pl.ds(s * ROWS_PT, ROWS_PT)])
    plsc.subcore_barrier()

    nh = NCHUNK // 2
    for part in range(2):
        pltpu.sync_copy(srcb_hbm.at[(c * 16 + s) * 2 + part], idx_s)
        pltpu.sync_copy(dstp_hbm.at[s * 2 + part], idx_d)
        for b in range(NBUF):
            pltpu.async_copy(g_hbm.at[idx_s.at[b]], rows[b], gsem[b])

        def group(j, carry):
            for b in range(NBUF):
                i = j * NBUF + b
                pltpu.make_async_copy(g_hbm.at[idx_s.at[i]], rows[b],
                                      gsem[b]).wait()
                pltpu.sync_copy(rows[b], acc_sh.at[idx_d.at[i]], add=True)

                @pl.when(j < nh // NBUF - 1)
                def _():
                    pltpu.async_copy(g_hbm.at[idx_s.at[i + NBUF]], rows[b],
                                     gsem[b])

            return carry

        lax.fori_loop(0, nh // NBUF, group, 0)
    plsc.subcore_barrier()

    off = s * ROWS_PT

    @pl.when(s < 15)
    def _():
        pltpu.sync_copy(acc_sh.at[pl.ds(off, ROWS_PT)],
                        out_hbm.at[pl.ds(c * N + off, ROWS_PT)])

    @pl.when(s == 15)
    def _():
        pltpu.sync_copy(acc_sh.at[pl.ds(off, N - 15 * ROWS_PT)],
                        out_hbm.at[pl.ds(c * N + off, N - 15 * ROWS_PT)])


@functools.partial(
    pl.kernel, mesh=_MESH,
    out_type=jax.ShapeDtypeStruct((32, 16), jnp.float32),
    scratch_types=[
        pltpu.VMEM((EPT_AD,), jnp.int32),
        pltpu.VMEM((EPT_AD,), jnp.int32),
        pltpu.VMEM((N_ACC,), jnp.float32),
        pltpu.VMEM((N_ACC,), jnp.float32),
        pltpu.VMEM((16,), jnp.float32),
    ],
    compiler_params=pltpu.CompilerParams(needs_layout_passes=False),
)
def _edge_dot_kernel(srcp_hbm, dstp_hbm, t_hbm, dinv_hbm, out_hbm,
                     src_v, dst_v, t_v, dinv_v, red_v):
    c = lax.axis_index("c")
    s = lax.axis_index("s")
    wid = s * 2 + c
    pltpu.sync_copy(t_hbm, t_v)
    pltpu.sync_copy(dinv_hbm, dinv_v)
    pltpu.sync_copy(srcp_hbm.at[pl.ds(wid * EPT_AD, EPT_AD)], src_v)
    pltpu.sync_copy(dstp_hbm.at[pl.ds(wid * EPT_AD, EPT_AD)], dst_v)

    def body(i, acc):
        si = src_v[pl.ds(i * 16, 16)]
        di = dst_v[pl.ds(i * 16, 16)]
        tv = plsc.load_gather(t_v, [si])
        dv = plsc.load_gather(dinv_v, [di])
        return acc + tv * dv

    acc = lax.fori_loop(0, EPT_AD // 16, body, jnp.zeros((16,), jnp.float32))
    red_v[...] = acc
    pltpu.sync_copy(red_v, out_hbm.at[wid])


_BLK = 1024


def _b1_body(x_ref, w_ref, cnt_ref, g_ref, dinv_ref):
    deg = jnp.sum(cnt_ref[...], axis=0) + 1.0
    dinv = lax.rsqrt(jnp.maximum(deg, 1e-12))[:, None]
    g = jnp.dot(x_ref[...], w_ref[...], preferred_element_type=jnp.float32)
    g = g * dinv
    g_ref[0] = g[:, :H]
    g_ref[1] = g[:, H:]
    dinv_ref[...] = dinv


def _b2_body(agg_ref, g_ref, dinv_ref, b_ref, w_ref, out_ref):
    dinv = dinv_ref[...]
    pre = jnp.concatenate([agg_ref[0] + g_ref[0], agg_ref[1] + g_ref[1]],
                          axis=1)
    h = jax.nn.relu(dinv * pre + b_ref[...][None, :])
    g2 = jnp.dot(h, w_ref[...], preferred_element_type=jnp.float32) * dinv
    out_ref[0] = g2[:, :H]
    out_ref[1] = g2[:, H:]


def _b3_body(agg_ref, g_ref, dinv_ref, b_ref, w_ref, t_ref, r_ref):
    dinv = dinv_ref[...]
    pre = jnp.concatenate([agg_ref[0] + g_ref[0], agg_ref[1] + g_ref[1]],
                          axis=1)
    h = jax.nn.relu(dinv * pre + b_ref[...][None, :])
    sv = jnp.dot(h, w_ref[...], preferred_element_type=jnp.float32)
    t_ref[...] = sv * dinv

    @pl.when(pl.program_id(0) == 0)
    def _():
        r_ref[...] = jnp.zeros_like(r_ref)

    row = pl.program_id(0) * _BLK + lax.broadcasted_iota(jnp.int32,
                                                         sv.shape, 0)
    r_ref[...] += jnp.sum(jnp.where(row < N, sv * dinv * dinv, 0.0))


def _split_spec(i):
    return (0, i, 0)


def kernel(x, edge_index, W1, b1, W2, b2, W3, b3):
    f32 = jnp.float32
    src = edge_index[0]
    dst = edge_index[1]
    npad = E_P - E
    srcp = jnp.concatenate([src, jnp.zeros((npad,), jnp.int32)])
    dstp = jnp.concatenate([dst, jnp.full((npad,), N, jnp.int32)])
    srcb = jnp.concatenate([srcp, srcp + N]).reshape(64, NCHUNK // 2, CHUNK)
    dstc = dstp.reshape(32, NCHUNK // 2, CHUNK)
    zeros_n = jnp.zeros((N_ACC,), f32)
    zrows = jnp.zeros((ROWS_PT, H), f32)

    cnt = _deg_kernel(dstp, zeros_n)

    grid = (N + _BLK - 1) // _BLK
    g_spec = pl.BlockSpec((2, _BLK, H), _split_spec)
    dinv_spec = pl.BlockSpec((_BLK, 1), lambda i: (i, 0))
    w_spec = pl.BlockSpec((D, D), lambda i: (0, 0))
    b_spec = pl.BlockSpec((D,), lambda i: (0,))

    g1, dinv = pl.pallas_call(
        _b1_body,
        grid=(grid,),
        in_specs=[
            pl.BlockSpec((_BLK, D), lambda i: (i, 0)),
            w_spec,
            pl.BlockSpec((32, _BLK), lambda i: (0, i)),
        ],
        out_specs=[g_spec, dinv_spec],
        out_shape=[
            jax.ShapeDtypeStruct((2, N, H), f32),
            jax.ShapeDtypeStruct((N, 1), f32),
        ],
    )(x, W1, cnt)

    g1f = g1.reshape(2 * N, H)
    agg1 = _agg_kernel(srcb, dstc, g1f, zrows).reshape(2, N, H)

    g2 = pl.pallas_call(
        _b2_body,
        grid=(grid,),
        in_specs=[g_spec, g_spec, dinv_spec, b_spec, w_spec],
        out_specs=g_spec,
        out_shape=jax.ShapeDtypeStruct((2, N, H), f32),
    )(agg1, g1, dinv, b1, W2)

    g2f = g2.reshape(2 * N, H)
    agg2 = _agg_kernel(srcb, dstc, g2f, zrows).reshape(2, N, H)

    t, r = pl.pallas_call(
        _b3_body,
        grid=(grid,),
        in_specs=[g_spec, g_spec, dinv_spec, b_spec,
                  pl.BlockSpec((D, 1), lambda i: (0, 0))],
        out_specs=[dinv_spec, pl.BlockSpec((1, 1), lambda i: (0, 0))],
        out_shape=[
            jax.ShapeDtypeStruct((N, 1), f32),
            jax.ShapeDtypeStruct((1, 1), f32),
        ],
    )(agg2, g2, dinv, b2, W3)

    tp = jnp.concatenate([t[:, 0], jnp.zeros((N_ACC - N,), f32)])
    dinvp = jnp.concatenate([dinv[:, 0], jnp.zeros((N_ACC - N,), f32)])
    parts = _edge_dot_kernel(srcp, dstp, tp, dinvp)

    out = jnp.sum(parts) + r[0, 0] + jnp.float32(N) * b3[0]
    return jnp.reshape(out, (1,))


# speedup vs baseline: 1.0042x; 1.0042x over previous
import functools
import jax
import jax.numpy as jnp
from jax import lax
from jax.experimental import pallas as pl
from jax.experimental.pallas import tpu as pltpu
from jax.experimental.pallas import tpu_sc as plsc

N = 10000
E = 160000
D = 256
H = 128
E_P = 163840
EPT_C = E_P // 16
EPT_AD = E_P // 32
N_ACC = 10112
ROWS_PT = N_ACC // 16
CHUNK = 128
NCHUNK = EPT_C // CHUNK
NBUF = 2

_MESH = plsc.VectorSubcoreMesh(core_axis_name="c", subcore_axis_name="s")


@functools.partial(
    pl.kernel, mesh=_MESH,
    out_type=jax.ShapeDtypeStruct((32, N_ACC), jnp.float32),
    scratch_types=[
        pltpu.VMEM((EPT_AD,), jnp.int32),
        pltpu.VMEM((N_ACC,), jnp.float32),
    ],
    compiler_params=pltpu.CompilerParams(needs_layout_passes=False),
)
def _deg_kernel(dst_hbm, zeros_hbm, out_hbm, dst_v, acc_v):
    c = lax.axis_index("c")
    s = lax.axis_index("s")
    wid = s * 2 + c
    pltpu.sync_copy(zeros_hbm, acc_v)
    pltpu.sync_copy(dst_hbm.at[pl.ds(wid * EPT_AD, EPT_AD)], dst_v)
    ones = jnp.full((16,), 1.0, jnp.float32)

    def body(i, carry):
        idx = dst_v[pl.ds(i * 16, 16)]
        plsc.addupdate_scatter(acc_v, [idx], ones)
        return carry

    lax.fori_loop(0, EPT_AD // 16, body, 0)
    pltpu.sync_copy(acc_v, out_hbm.at[wid])


@functools.partial(
    pl.kernel, mesh=_MESH,
    out_type=jax.ShapeDtypeStruct((2 * N, H), jnp.float32),
    scratch_types=[
        pltpu.VMEM((NCHUNK // 2, CHUNK), jnp.int32),
        pltpu.VMEM((NCHUNK // 2, CHUNK), jnp.int32),
        [pltpu.VMEM((CHUNK, H), jnp.float32)] * NBUF,
        [pltpu.SemaphoreType.DMA] * NBUF,
        pltpu.VMEM_SHARED((N_ACC, H), jnp.float32),
    ],
)
def _agg_kernel(srcb_hbm, dstp_hbm, g_hbm, zrows_hbm, out_hbm,
                idx_s, idx_d, rows, gsem, acc_sh):
    c = lax.axis_index("c")
    s = lax.axis_index("s")
    zcp = pltpu.async_copy(zrows_hbm, acc_sh.at[pl.ds(s * ROWS_PT, ROWS_PT)],
                           gsem[0])
    pltpu.sync_copy(srcb_hbm.at[(c * 16 + s) * 2], idx_s)
    pltpu.sync_copy(dstp_hbm.at[s * 2], idx_d)
    zcp.wait()
    plsc.subcore_barrier()

    nh = NCHUNK // 2
    for part in range(2):
        if part:
            pltpu.sync_copy(srcb_hbm.at[(c * 16 + s) * 2 + part], idx_s)
            pltpu.sync_copy(dstp_hbm.at[s * 2 + part], idx_d)
        for b in range(NBUF):
            pltpu.async_copy(g_hbm.at[idx_s.at[b]], rows[b], gsem[b])

        def group(j, carry):
            for b in range(NBUF):
                i = j * NBUF + b
                pltpu.make_async_copy(g_hbm.at[idx_s.at[i]], rows[b],
                                      gsem[b]).wait()
                pltpu.sync_copy(rows[b], acc_sh.at[idx_d.at[i]], add=True)

                @pl.when(j < nh // NBUF - 1)
                def _():
                    pltpu.async_copy(g_hbm.at[idx_s.at[i + NBUF]], rows[b],
                                     gsem[b])

            return carry

        lax.fori_loop(0, nh // NBUF, group, 0)
    plsc.subcore_barrier()

    off = s * ROWS_PT

    @pl.when(s < 15)
    def _():
        pltpu.sync_copy(acc_sh.at[pl.ds(off, ROWS_PT)],
                        out_hbm.at[pl.ds(c * N + off, ROWS_PT)])

    @pl.when(s == 15)
    def _():
        pltpu.sync_copy(acc_sh.at[pl.ds(off, N - 15 * ROWS_PT)],
                        out_hbm.at[pl.ds(c * N + off, N - 15 * ROWS_PT)])


@functools.partial(
    pl.kernel, mesh=_MESH,
    out_type=jax.ShapeDtypeStruct((32, 16), jnp.float32),
    scratch_types=[
        pltpu.VMEM((EPT_AD,), jnp.int32),
        pltpu.VMEM((EPT_AD,), jnp.int32),
        pltpu.VMEM((N_ACC,), jnp.float32),
        pltpu.VMEM((N_ACC,), jnp.float32),
        pltpu.VMEM((16,), jnp.float32),
    ],
    compiler_params=pltpu.CompilerParams(needs_layout_passes=False),
)
def _edge_dot_kernel(srcp_hbm, dstp_hbm, t_hbm, dinv_hbm, out_hbm,
                     src_v, dst_v, t_v, dinv_v, red_v):
    c = lax.axis_index("c")
    s = lax.axis_index("s")
    wid = s * 2 + c
    pltpu.sync_copy(t_hbm, t_v)
    pltpu.sync_copy(dinv_hbm, dinv_v)
    pltpu.sync_copy(srcp_hbm.at[pl.ds(wid * EPT_AD, EPT_AD)], src_v)
    pltpu.sync_copy(dstp_hbm.at[pl.ds(wid * EPT_AD, EPT_AD)], dst_v)

    def body(i, acc):
        si = src_v[pl.ds(i * 16, 16)]
        di = dst_v[pl.ds(i * 16, 16)]
        tv = plsc.load_gather(t_v, [si])
        dv = plsc.load_gather(dinv_v, [di])
        return acc + tv * dv

    acc = lax.fori_loop(0, EPT_AD // 16, body, jnp.zeros((16,), jnp.float32))
    red_v[...] = acc
    pltpu.sync_copy(red_v, out_hbm.at[wid])


_BLK = 1024


def _b1_body(x_ref, w_ref, cnt_ref, g_ref, dinv_ref):
    deg = jnp.sum(cnt_ref[...], axis=0) + 1.0
    dinv = lax.rsqrt(jnp.maximum(deg, 1e-12))[:, None]
    g = jnp.dot(x_ref[...], w_ref[...], preferred_element_type=jnp.float32)
    g = g * dinv
    g_ref[0] = g[:, :H]
    g_ref[1] = g[:, H:]
    dinv_ref[...] = dinv


def _b2_body(agg_ref, g_ref, dinv_ref, b_ref, w_ref, out_ref):
    dinv = dinv_ref[...]
    pre = jnp.concatenate([agg_ref[0] + g_ref[0], agg_ref[1] + g_ref[1]],
                          axis=1)
    h = jax.nn.relu(dinv * pre + b_ref[...][None, :])
    g2 = jnp.dot(h, w_ref[...], preferred_element_type=jnp.float32) * dinv
    out_ref[0] = g2[:, :H]
    out_ref[1] = g2[:, H:]


def _b3_body(agg_ref, g_ref, dinv_ref, b_ref, w_ref, t_ref, r_ref):
    dinv = dinv_ref[...]
    pre = jnp.concatenate([agg_ref[0] + g_ref[0], agg_ref[1] + g_ref[1]],
                          axis=1)
    h = jax.nn.relu(dinv * pre + b_ref[...][None, :])
    sv = jnp.dot(h, w_ref[...], preferred_element_type=jnp.float32)
    t_ref[...] = sv * dinv

    @pl.when(pl.program_id(0) == 0)
    def _():
        r_ref[...] = jnp.zeros_like(r_ref)

    row = pl.program_id(0) * _BLK + lax.broadcasted_iota(jnp.int32,
                                                         sv.shape, 0)
    r_ref[...] += jnp.sum(jnp.where(row < N, sv * dinv * dinv, 0.0))


def _split_spec(i):
    return (0, i, 0)


def kernel(x, edge_index, W1, b1, W2, b2, W3, b3):
    f32 = jnp.float32
    src = edge_index[0]
    dst = edge_index[1]
    npad = E_P - E
    srcp = jnp.concatenate([src, jnp.zeros((npad,), jnp.int32)])
    dstp = jnp.concatenate([dst, jnp.full((npad,), N, jnp.int32)])
    srcb = jnp.concatenate([srcp, srcp + N]).reshape(64, NCHUNK // 2, CHUNK)
    dstc = dstp.reshape(32, NCHUNK // 2, CHUNK)
    zeros_n = jnp.zeros((N_ACC,), f32)
    zrows = jnp.zeros((ROWS_PT, H), f32)

    cnt = _deg_kernel(dstp, zeros_n)

    grid = (N + _BLK - 1) // _BLK
    g_spec = pl.BlockSpec((2, _BLK, H), _split_spec)
    dinv_spec = pl.BlockSpec((_BLK, 1), lambda i: (i, 0))
    w_spec = pl.BlockSpec((D, D), lambda i: (0, 0))
    b_spec = pl.BlockSpec((D,), lambda i: (0,))

    g1, dinv = pl.pallas_call(
        _b1_body,
        grid=(grid,),
        in_specs=[
            pl.BlockSpec((_BLK, D), lambda i: (i, 0)),
            w_spec,
            pl.BlockSpec((32, _BLK), lambda i: (0, i)),
        ],
        out_specs=[g_spec, dinv_spec],
        out_shape=[
            jax.ShapeDtypeStruct((2, N, H), f32),
            jax.ShapeDtypeStruct((N, 1), f32),
        ],
    )(x, W1, cnt)

    g1f = g1.reshape(2 * N, H)
    agg1 = _agg_kernel(srcb, dstc, g1f, zrows).reshape(2, N, H)

    g2 = pl.pallas_call(
        _b2_body,
        grid=(grid,),
        in_specs=[g_spec, g_spec, dinv_spec, b_spec, w_spec],
        out_specs=g_spec,
        out_shape=jax.ShapeDtypeStruct((2, N, H), f32),
    )(agg1, g1, dinv, b1, W2)

    g2f = g2.reshape(2 * N, H)
    agg2 = _agg_kernel(srcb, dstc, g2f, zrows).reshape(2, N, H)

    t, r = pl.pallas_call(
        _b3_body,
        grid=(grid,),
        in_specs=[g_spec, g_spec, dinv_spec, b_spec,
                  pl.BlockSpec((D, 1), lambda i: (0, 0))],
        out_specs=[dinv_spec, pl.BlockSpec((1, 1), lambda i: (0, 0))],
        out_shape=[
            jax.ShapeDtypeStruct((N, 1), f32),
            jax.ShapeDtypeStruct((1, 1), f32),
        ],
    )(agg2, g2, dinv, b2, W3)

    tp = jnp.concatenate([t[:, 0], jnp.zeros((N_ACC - N,), f32)])
    dinvp = jnp.concatenate([dinv[:, 0], jnp.zeros((N_ACC - N,), f32)])
    parts = _edge_dot_kernel(srcp, dstp, tp, dinvp)

    out = jnp.sum(parts) + r[0, 0] + jnp.float32(N) * b3[0]
    return jnp.reshape(out, (1,))
